# Initial kernel scaffold; baseline (speedup 1.0000x reference)
import jax
import jax.numpy as jnp
from jax.experimental import pallas as pl


def kernel(indices, table):
    raise NotImplementedError("write your pallas kernel here")


# speedup vs baseline: 6.1817x; 6.1817x over previous
import functools

import jax
import jax.numpy as jnp
from jax import lax
from jax.experimental import pallas as pl
from jax.experimental.pallas import tpu as pltpu
from jax.experimental.pallas import tpu_sc as plsc

_D = 64


@functools.lru_cache(maxsize=None)
def _build_gather(B, C):
    info = plsc.get_sparse_core_info()
    NC, NS = info.num_cores, info.num_subcores
    NW = NC * NS
    assert B % (NW * C) == 0
    b_per_w = B // NW
    nt = b_per_w // C
    mesh = plsc.VectorSubcoreMesh(core_axis_name="c", subcore_axis_name="s")

    @functools.partial(
        pl.kernel,
        mesh=mesh,
        out_type=jax.ShapeDtypeStruct((B, _D), jnp.float32),
        scratch_types=[
            pltpu.VMEM((C,), jnp.int32),
            pltpu.VMEM((C, _D), jnp.float32),
            pltpu.SemaphoreType.DMA,
        ],
        compiler_params=pltpu.CompilerParams(use_tc_tiling_on_sc=False),
    )
    def gather_kernel(idx_hbm, table_hbm, out_hbm, idx_v, rows_v, sem):
        wid = lax.axis_index("s") * NC + lax.axis_index("c")
        base = wid * b_per_w

        def body(t, carry):
            off = base + t * C
            pltpu.sync_copy(idx_hbm.at[pl.ds(off, C)], idx_v)
            pltpu.async_copy(table_hbm.at[idx_v], rows_v, sem).wait()
            pltpu.sync_copy(rows_v, out_hbm.at[pl.ds(off, C)])
            return carry

        lax.fori_loop(0, nt, body, 0)

    return gather_kernel


def kernel(indices, table):
    B0, H = indices.shape
    flat = indices.reshape(-1).astype(jnp.int32)
    out = _build_gather(flat.shape[0], 1600)(flat, table)
    return out.reshape(B0, H, _D)


# speedup vs baseline: 6.2520x; 1.0114x over previous
import functools

import jax
import jax.numpy as jnp
from jax import lax
from jax.experimental import pallas as pl
from jax.experimental.pallas import tpu as pltpu
from jax.experimental.pallas import tpu_sc as plsc

_D = 64


@functools.lru_cache(maxsize=None)
def _build_gather(B, C):
    info = plsc.get_sparse_core_info()
    NC, NS = info.num_cores, info.num_subcores
    NW = NC * NS
    assert B % (NW * C) == 0
    b_per_w = B // NW
    nt = b_per_w // C
    assert nt % 2 == 0
    mesh = plsc.VectorSubcoreMesh(core_axis_name="c", subcore_axis_name="s")

    @functools.partial(
        pl.kernel,
        mesh=mesh,
        out_type=jax.ShapeDtypeStruct((B, _D), jnp.float32),
        scratch_types=[
            pltpu.VMEM((2, C), jnp.int32),
            pltpu.VMEM((2, C, _D), jnp.float32),
            pltpu.SemaphoreType.DMA,
            pltpu.SemaphoreType.DMA,
            pltpu.SemaphoreType.DMA,
            pltpu.SemaphoreType.DMA,
        ],
        compiler_params=pltpu.CompilerParams(use_tc_tiling_on_sc=False),
    )
    def gather_kernel(idx_hbm, table_hbm, out_hbm, idx_v, rows_v, g0, g1, o0, o1):
        gsem = (g0, g1)
        osem = (o0, o1)
        wid = lax.axis_index("s") * NC + lax.axis_index("c")
        base = wid * b_per_w

        def idx_load(b, off):
            pltpu.sync_copy(idx_hbm.at[pl.ds(off, C)], idx_v.at[b])

        def gather_desc(b):
            return pltpu.make_async_copy(
                table_hbm.at[idx_v.at[b]], rows_v.at[b], gsem[b])

        def out_desc(b, off):
            return pltpu.make_async_copy(
                rows_v.at[b], out_hbm.at[pl.ds(off, C)], osem[b])

        for b in range(2):
            idx_load(b, base + b * C)
            gather_desc(b).start()

        npairs = nt // 2

        def body(tt, carry):
            for b in range(2):
                off = base + (tt * 2 + b) * C
                gather_desc(b).wait()
                out_desc(b, off).start()
                idx_load(b, off + 2 * C)
                out_desc(b, off).wait()
                gather_desc(b).start()
            return carry

        lax.fori_loop(0, npairs - 1, body, 0)

        for b in range(2):
            off = base + ((npairs - 1) * 2 + b) * C
            gather_desc(b).wait()
            out_desc(b, off).start()
        for b in range(2):
            off = base + ((npairs - 1) * 2 + b) * C
            out_desc(b, off).wait()

    return gather_kernel


def kernel(indices, table):
    B0, H = indices.shape
    flat = indices.reshape(-1).astype(jnp.int32)
    out = _build_gather(flat.shape[0], 800)(flat, table)
    return out.reshape(B0, H, _D)
